# baseline (speedup 1.0000x reference)
import functools

import jax
import jax.numpy as jnp
from jax import lax
from jax.experimental import pallas as pl
from jax.experimental.pallas import tpu as pltpu
from jax.experimental.pallas import tpu_sc as plsc

STATE = 128
N_ROWS = 100000
BLK = 10000
NSTEPS = N_ROWS // BLK

NL_SC = 60000
BLKL = (N_ROWS - NL_SC) // NSTEPS
LOFF = NL_SC // BLKL

NW = 32
NACT = 30
ROWS_W = NL_SC // NACT
CH = 250
NCH = ROWS_W // CH
NBUF = 3


def _sc_ltm_body(emb_hbm, w_hbm, out_hbm, rows0, rows1, rows2, w_v, acc_v,
                 *sems):
    c = lax.axis_index("c")
    s = lax.axis_index("s")
    wid = s * 2 + c
    bufs = (rows0, rows1, rows2)

    @pl.when(wid < NACT)
    def _work():
        base = wid * ROWS_W
        pltpu.sync_copy(w_hbm.at[pl.ds(base, ROWS_W)],
                        w_v.at[pl.ds(0, ROWS_W)])
        for b in range(NBUF - 1):
            pltpu.make_async_copy(
                emb_hbm.at[pl.ds((base + b * CH) * STATE, CH * STATE)],
                bufs[b], sems[b]).start()

        accs = tuple(jnp.zeros((16,), jnp.float32) for _ in range(8))
        for ci in range(NCH):
            b = ci % NBUF
            nxt = ci + NBUF - 1
            if nxt < NCH:
                nb = nxt % NBUF
                pltpu.make_async_copy(
                    emb_hbm.at[pl.ds((base + nxt * CH) * STATE, CH * STATE)],
                    bufs[nb], sems[nb]).start()
            pltpu.make_async_copy(
                emb_hbm.at[pl.ds((base + ci * CH) * STATE, CH * STATE)],
                bufs[b], sems[b]).wait()

            def row_body(i, accs, ci=ci, b=b):
                wv = w_v[pl.ds(ci * CH + i, 16)][0]
                return tuple(
                    accs[k] + wv * bufs[b][pl.ds(i * STATE + 16 * k, 16)]
                    for k in range(8))

            accs = lax.fori_loop(0, CH, row_body, accs)

        for k in range(8):
            acc_v[pl.ds(16 * k, 16)] = accs[k]
        pltpu.sync_copy(acc_v, out_hbm.at[pl.ds(wid * STATE, STATE)])

    @pl.when(wid >= NACT)
    def _idle():
        z = jnp.zeros((16,), jnp.float32)
        for k in range(8):
            acc_v[pl.ds(16 * k, 16)] = z
        pltpu.sync_copy(acc_v, out_hbm.at[pl.ds(wid * STATE, STATE)])


_sc_ltm = functools.partial(
    pl.kernel,
    out_type=jax.ShapeDtypeStruct((NW * STATE,), jnp.float32),
    mesh=plsc.VectorSubcoreMesh(core_axis_name="c", subcore_axis_name="s"),
    scratch_types=[
        pltpu.VMEM((CH * STATE,), jnp.float32),
        pltpu.VMEM((CH * STATE,), jnp.float32),
        pltpu.VMEM((CH * STATE,), jnp.float32),
        pltpu.VMEM((ROWS_W + 16,), jnp.float32),
        pltpu.VMEM((STATE,), jnp.float32),
    ] + [pltpu.SemaphoreType.DMA] * NBUF,
)(_sc_ltm_body)


def _tc_main_body(x_ref, stm_ref, stmw_ref, ltm_ref, ltmw_ref,
                  rs_ref, lp_ref, accs_ref, accl_ref, m_ref, s_ref):
    j = pl.program_id(0)

    @pl.when(j == 0)
    def _init():
        accs_ref[...] = jnp.zeros_like(accs_ref)
        accl_ref[...] = jnp.zeros_like(accl_ref)
        m_ref[0] = jnp.float32(-1e30)
        s_ref[0] = jnp.float32(0.0)

    x = x_ref[...]
    stm = stm_ref[...]
    w = stmw_ref[...].reshape(1, BLK)

    scores = jax.lax.dot_general(
        x, stm, (((1,), (1,)), ((), ())),
        preferred_element_type=jnp.float32)
    t = scores * w
    bm = jnp.max(t)
    m_old = m_ref[0]
    m_new = jnp.maximum(m_old, bm)
    c = jnp.exp(m_old - m_new)
    p = jnp.exp(t - m_new)
    s_ref[0] = s_ref[0] * c + jnp.sum(p)
    accs_ref[...] = accs_ref[...] * c + jax.lax.dot_general(
        p, stm, (((1,), (0,)), ((), ())),
        preferred_element_type=jnp.float32)
    m_ref[0] = m_new

    ltm = ltm_ref[...]
    lw = ltmw_ref[...].reshape(1, BLKL)
    accl_ref[...] += jax.lax.dot_general(
        lw, ltm, (((1,), (0,)), ((), ())),
        preferred_element_type=jnp.float32)

    @pl.when(j == NSTEPS - 1)
    def _fin():
        rs_ref[...] = accs_ref[...] / s_ref[0]
        lp_ref[...] = accl_ref[...]


def _tc_main(x2d, stm_emb, stm_w3d, ltm_emb, ltm_w3d):
    return pl.pallas_call(
        _tc_main_body,
        grid=(NSTEPS,),
        in_specs=[
            pl.BlockSpec((1, STATE), lambda j: (0, 0)),
            pl.BlockSpec((BLK, STATE), lambda j: (j, 0)),
            pl.BlockSpec((1, 1, BLK), lambda j: (j, 0, 0)),
            pl.BlockSpec((BLKL, STATE), lambda j: (j + LOFF, 0)),
            pl.BlockSpec((1, 1, BLKL), lambda j: (j + LOFF, 0, 0)),
        ],
        out_specs=[
            pl.BlockSpec((1, STATE), lambda j: (0, 0)),
            pl.BlockSpec((1, STATE), lambda j: (0, 0)),
        ],
        out_shape=[
            jax.ShapeDtypeStruct((1, STATE), jnp.float32),
            jax.ShapeDtypeStruct((1, STATE), jnp.float32),
        ],
        scratch_shapes=[
            pltpu.VMEM((1, STATE), jnp.float32),
            pltpu.VMEM((1, STATE), jnp.float32),
            pltpu.SMEM((1,), jnp.float32),
            pltpu.SMEM((1,), jnp.float32),
        ],
    )(x2d, stm_emb, stm_w3d, ltm_emb, ltm_w3d)


def _tc_mlp_body(x_ref, rs_ref, tclp_ref, sclp_ref, W1_ref, b1_ref,
                 W2_ref, b2_ref, out_ref):
    r_l = tclp_ref[...] + jnp.sum(sclp_ref[...], axis=0, keepdims=True)
    fused = jnp.concatenate([x_ref[...], rs_ref[...], r_l], axis=1)
    h = jnp.maximum(
        jnp.dot(fused, W1_ref[...], preferred_element_type=jnp.float32)
        + b1_ref[...], 0.0)
    out_ref[...] = (
        jnp.dot(h, W2_ref[...], preferred_element_type=jnp.float32)
        + b2_ref[...])


def _tc_mlp(x2d, rs, tclp, sclp, W1, b1_2d, W2, b2_2d):
    return pl.pallas_call(
        _tc_mlp_body,
        out_shape=jax.ShapeDtypeStruct((1, STATE), jnp.float32),
    )(x2d, rs, tclp, sclp, W1, b1_2d, W2, b2_2d)


@jax.jit
def kernel(x_t, stm_emb, stm_weight, ltm_emb, ltm_weight, W1, b1, W2, b2):
    x2d = x_t.reshape(1, STATE)
    sclp = _sc_ltm(ltm_emb.reshape(N_ROWS * STATE), ltm_weight)
    rs, tclp = _tc_main(
        x2d, stm_emb, stm_weight.reshape(NSTEPS, 1, BLK),
        ltm_emb, ltm_weight.reshape(N_ROWS // BLKL, 1, BLKL))
    out = _tc_mlp(x2d, rs, tclp, sclp.reshape(NW, STATE), W1,
                  b1.reshape(1, STATE), W2, b2.reshape(1, STATE))
    return out.reshape(STATE)


# speedup vs baseline: 1.0105x; 1.0105x over previous
import functools

import jax
import jax.numpy as jnp
from jax import lax
from jax.experimental import pallas as pl
from jax.experimental.pallas import tpu as pltpu
from jax.experimental.pallas import tpu_sc as plsc

STATE = 128
N_ROWS = 100000
BLK = 10000
NSTEPS = N_ROWS // BLK

NL_SC = 60000
BLKL = (N_ROWS - NL_SC) // NSTEPS
LOFF = NL_SC // BLKL

NW = 32
NACT = 30
ROWS_W = NL_SC // NACT
CH = 250
NCH = ROWS_W // CH
NBUF = 3


def _sc_ltm_body(emb_hbm, w_hbm, out_hbm, rows0, rows1, rows2, w_v, acc_v,
                 *sems):
    c = lax.axis_index("c")
    s = lax.axis_index("s")
    wid = s * 2 + c
    bufs = (rows0, rows1, rows2)

    @pl.when(wid < NACT)
    def _work():
        base = wid * ROWS_W
        pltpu.sync_copy(w_hbm.at[pl.ds(base, ROWS_W)],
                        w_v.at[pl.ds(0, ROWS_W)])
        for b in range(NBUF - 1):
            pltpu.make_async_copy(
                emb_hbm.at[pl.ds((base + b * CH) * STATE, CH * STATE)],
                bufs[b], sems[b]).start()

        accs = tuple(jnp.zeros((16,), jnp.float32) for _ in range(8))
        for ci in range(NCH):
            b = ci % NBUF
            nxt = ci + NBUF - 1
            if nxt < NCH:
                nb = nxt % NBUF
                pltpu.make_async_copy(
                    emb_hbm.at[pl.ds((base + nxt * CH) * STATE, CH * STATE)],
                    bufs[nb], sems[nb]).start()
            pltpu.make_async_copy(
                emb_hbm.at[pl.ds((base + ci * CH) * STATE, CH * STATE)],
                bufs[b], sems[b]).wait()

            def row_body(i, accs, ci=ci, b=b):
                wv = w_v[pl.ds(ci * CH + i, 16)][0]
                return tuple(
                    accs[k] + wv * bufs[b][pl.ds(i * STATE + 16 * k, 16)]
                    for k in range(8))

            accs = lax.fori_loop(0, CH, row_body, accs)

        for k in range(8):
            acc_v[pl.ds(16 * k, 16)] = accs[k]
        pltpu.sync_copy(acc_v, out_hbm.at[pl.ds(wid * STATE, STATE)])

    @pl.when(wid >= NACT)
    def _idle():
        z = jnp.zeros((16,), jnp.float32)
        for k in range(8):
            acc_v[pl.ds(16 * k, 16)] = z
        pltpu.sync_copy(acc_v, out_hbm.at[pl.ds(wid * STATE, STATE)])


_sc_ltm = functools.partial(
    pl.kernel,
    out_type=jax.ShapeDtypeStruct((NW * STATE,), jnp.float32),
    mesh=plsc.VectorSubcoreMesh(core_axis_name="c", subcore_axis_name="s"),
    compiler_params=pltpu.CompilerParams(skip_device_barrier=True),
    scratch_types=[
        pltpu.VMEM((CH * STATE,), jnp.float32),
        pltpu.VMEM((CH * STATE,), jnp.float32),
        pltpu.VMEM((CH * STATE,), jnp.float32),
        pltpu.VMEM((ROWS_W + 16,), jnp.float32),
        pltpu.VMEM((STATE,), jnp.float32),
    ] + [pltpu.SemaphoreType.DMA] * NBUF,
)(_sc_ltm_body)


def _tc_main_body(x_ref, stm_ref, stmw_ref, ltm_ref, ltmw_ref,
                  rs_ref, lp_ref, accs_ref, accl_ref, m_ref, s_ref):
    j = pl.program_id(0)

    @pl.when(j == 0)
    def _init():
        accs_ref[...] = jnp.zeros_like(accs_ref)
        accl_ref[...] = jnp.zeros_like(accl_ref)
        m_ref[0] = jnp.float32(-1e30)
        s_ref[0] = jnp.float32(0.0)

    x = x_ref[...]
    stm = stm_ref[...]
    w = stmw_ref[...].reshape(1, BLK)

    scores = jax.lax.dot_general(
        x, stm, (((1,), (1,)), ((), ())),
        preferred_element_type=jnp.float32)
    t = scores * w
    bm = jnp.max(t)
    m_old = m_ref[0]
    m_new = jnp.maximum(m_old, bm)
    c = jnp.exp(m_old - m_new)
    p = jnp.exp(t - m_new)
    s_ref[0] = s_ref[0] * c + jnp.sum(p)
    accs_ref[...] = accs_ref[...] * c + jax.lax.dot_general(
        p, stm, (((1,), (0,)), ((), ())),
        preferred_element_type=jnp.float32)
    m_ref[0] = m_new

    ltm = ltm_ref[...]
    lw = ltmw_ref[...].reshape(1, BLKL)
    accl_ref[...] += jax.lax.dot_general(
        lw, ltm, (((1,), (0,)), ((), ())),
        preferred_element_type=jnp.float32)

    @pl.when(j == NSTEPS - 1)
    def _fin():
        rs_ref[...] = accs_ref[...] / s_ref[0]
        lp_ref[...] = accl_ref[...]


def _tc_main(x2d, stm_emb, stm_w3d, ltm_emb, ltm_w3d):
    return pl.pallas_call(
        _tc_main_body,
        grid=(NSTEPS,),
        in_specs=[
            pl.BlockSpec((1, STATE), lambda j: (0, 0)),
            pl.BlockSpec((BLK, STATE), lambda j: (j, 0)),
            pl.BlockSpec((1, 1, BLK), lambda j: (j, 0, 0)),
            pl.BlockSpec((BLKL, STATE), lambda j: (j + LOFF, 0)),
            pl.BlockSpec((1, 1, BLKL), lambda j: (j + LOFF, 0, 0)),
        ],
        out_specs=[
            pl.BlockSpec((1, STATE), lambda j: (0, 0)),
            pl.BlockSpec((1, STATE), lambda j: (0, 0)),
        ],
        out_shape=[
            jax.ShapeDtypeStruct((1, STATE), jnp.float32),
            jax.ShapeDtypeStruct((1, STATE), jnp.float32),
        ],
        scratch_shapes=[
            pltpu.VMEM((1, STATE), jnp.float32),
            pltpu.VMEM((1, STATE), jnp.float32),
            pltpu.SMEM((1,), jnp.float32),
            pltpu.SMEM((1,), jnp.float32),
        ],
    )(x2d, stm_emb, stm_w3d, ltm_emb, ltm_w3d)


def _tc_mlp_body(x_ref, rs_ref, tclp_ref, sclp_ref, W1_ref, b1_ref,
                 W2_ref, b2_ref, out_ref):
    r_l = tclp_ref[...] + jnp.sum(sclp_ref[...], axis=0, keepdims=True)
    fused = jnp.concatenate([x_ref[...], rs_ref[...], r_l], axis=1)
    h = jnp.maximum(
        jnp.dot(fused, W1_ref[...], preferred_element_type=jnp.float32)
        + b1_ref[...], 0.0)
    out_ref[...] = (
        jnp.dot(h, W2_ref[...], preferred_element_type=jnp.float32)
        + b2_ref[...])


def _tc_mlp(x2d, rs, tclp, sclp, W1, b1_2d, W2, b2_2d):
    return pl.pallas_call(
        _tc_mlp_body,
        out_shape=jax.ShapeDtypeStruct((1, STATE), jnp.float32),
    )(x2d, rs, tclp, sclp, W1, b1_2d, W2, b2_2d)


@jax.jit
def kernel(x_t, stm_emb, stm_weight, ltm_emb, ltm_weight, W1, b1, W2, b2):
    x2d = x_t.reshape(1, STATE)
    sclp = _sc_ltm(ltm_emb.reshape(N_ROWS * STATE), ltm_weight)
    rs, tclp = _tc_main(
        x2d, stm_emb, stm_weight.reshape(NSTEPS, 1, BLK),
        ltm_emb, ltm_weight.reshape(N_ROWS // BLKL, 1, BLKL))
    out = _tc_mlp(x2d, rs, tclp, sclp.reshape(NW, STATE), W1,
                  b1.reshape(1, STATE), W2, b2.reshape(1, STATE))
    return out.reshape(STATE)


# speedup vs baseline: 1.3355x; 1.3217x over previous
import functools

import jax
import jax.numpy as jnp
from jax.experimental import pallas as pl
from jax.experimental.pallas import tpu as pltpu

STATE = 128
N_ROWS = 100000
BLK = 20000


def _body(x_ref, stm_ref, stmw_ref, ltm_ref, ltmw_ref,
          W1_ref, b1_ref, W2_ref, b2_ref, out_ref,
          accs_ref, accl_ref, m_ref, s_ref):
    j = pl.program_id(0)
    nsteps = pl.num_programs(0)

    @pl.when(j == 0)
    def _init():
        accs_ref[...] = jnp.zeros_like(accs_ref)
        accl_ref[...] = jnp.zeros_like(accl_ref)
        m_ref[0] = jnp.float32(-1e30)
        s_ref[0] = jnp.float32(0.0)

    x = x_ref[...]
    stm = stm_ref[...]
    w = stmw_ref[...].reshape(1, BLK)

    scores = jax.lax.dot_general(
        x, stm, (((1,), (1,)), ((), ())),
        preferred_element_type=jnp.float32)
    t = scores * w
    bm = jnp.max(t)
    m_old = m_ref[0]
    m_new = jnp.maximum(m_old, bm)
    c = jnp.exp(m_old - m_new)
    p = jnp.exp(t - m_new)
    s_ref[0] = s_ref[0] * c + jnp.sum(p)
    accs_ref[...] = accs_ref[...] * c + jax.lax.dot_general(
        p, stm, (((1,), (0,)), ((), ())),
        preferred_element_type=jnp.float32)
    m_ref[0] = m_new

    ltm = ltm_ref[...]
    lw = ltmw_ref[...].reshape(1, BLK)
    accl_ref[...] += jax.lax.dot_general(
        lw, ltm, (((1,), (0,)), ((), ())),
        preferred_element_type=jnp.float32)

    @pl.when(j == nsteps - 1)
    def _fin():
        r_s = accs_ref[...] / s_ref[0]
        fused = jnp.concatenate([x, r_s, accl_ref[...]], axis=1)
        h = jnp.maximum(
            jnp.dot(fused, W1_ref[...], preferred_element_type=jnp.float32)
            + b1_ref[...], 0.0)
        out_ref[...] = (
            jnp.dot(h, W2_ref[...], preferred_element_type=jnp.float32)
            + b2_ref[...])


@jax.jit
def kernel(x_t, stm_emb, stm_weight, ltm_emb, ltm_weight, W1, b1, W2, b2):
    nsteps = N_ROWS // BLK
    out = pl.pallas_call(
        _body,
        grid=(nsteps,),
        in_specs=[
            pl.BlockSpec((1, STATE), lambda j: (0, 0)),
            pl.BlockSpec((BLK, STATE), lambda j: (j, 0)),
            pl.BlockSpec((1, 1, BLK), lambda j: (j, 0, 0)),
            pl.BlockSpec((BLK, STATE), lambda j: (j, 0)),
            pl.BlockSpec((1, 1, BLK), lambda j: (j, 0, 0)),
            pl.BlockSpec((3 * STATE, STATE), lambda j: (0, 0)),
            pl.BlockSpec((1, STATE), lambda j: (0, 0)),
            pl.BlockSpec((STATE, STATE), lambda j: (0, 0)),
            pl.BlockSpec((1, STATE), lambda j: (0, 0)),
        ],
        out_specs=pl.BlockSpec((1, STATE), lambda j: (0, 0)),
        out_shape=jax.ShapeDtypeStruct((1, STATE), jnp.float32),
        scratch_shapes=[
            pltpu.VMEM((1, STATE), jnp.float32),
            pltpu.VMEM((1, STATE), jnp.float32),
            pltpu.SMEM((1,), jnp.float32),
            pltpu.SMEM((1,), jnp.float32),
        ],
    )(
        x_t.reshape(1, STATE), stm_emb, stm_weight.reshape(nsteps, 1, BLK),
        ltm_emb, ltm_weight.reshape(nsteps, 1, BLK),
        W1, b1.reshape(1, STATE), W2, b2.reshape(1, STATE),
    )
    return out.reshape(STATE)
